# baseline (speedup 1.0000x reference)
import dataclasses
import functools

import jax
import jax.numpy as jnp
from jax import lax
from jax.experimental import pallas as pl
from jax.experimental.pallas import tpu as pltpu
from jax.experimental.pallas import tpu_sc as plsc

N_ = 10000
D_ = 128
E_ = 320000
NC_ = 2
NS_ = 16
NW_ = NC_ * NS_
EPT_ = E_ // NW_
CH_ = 80
CHA_ = 48
CHB_ = 32
NCHUNK_ = EPT_ // CH_
RPT_ = 624
REM_ = N_ - NS_ * RPT_
ZR_ = 208
SR_ = 80
BLK_ = 2000


def _dot(a, b):
    return lax.dot_general(a, b, (((1,), (0,)), ((), ())),
                           precision=lax.Precision.HIGHEST)


def _proj(h, al_ref, ar_ref, el_ref, er_ref):
    el_ref[...] = jnp.sum(h * al_ref[...], axis=1, keepdims=True)
    er_ref[...] = jnp.sum(h * ar_ref[...], axis=1, keepdims=True)


def _tc_head_body(x_ref, w_ref, al_ref, ar_ref, h_ref, el_ref, er_ref):
    h = _dot(x_ref[...], w_ref[...])
    h_ref[...] = h
    _proj(h, al_ref, ar_ref, el_ref, er_ref)


_HEL_OUT = dict(
    out_specs=[pl.BlockSpec((BLK_, D_), lambda i: (i, 0)),
               pl.BlockSpec((BLK_, 1), lambda i: (i, 0)),
               pl.BlockSpec((BLK_, 1), lambda i: (i, 0))],
    out_shape=[jax.ShapeDtypeStruct((N_, D_), jnp.float32),
               jax.ShapeDtypeStruct((N_, 1), jnp.float32),
               jax.ShapeDtypeStruct((N_, 1), jnp.float32)],
)


def _tc_head(x, W, al, ar):
    return pl.pallas_call(
        _tc_head_body,
        grid=(N_ // BLK_,),
        in_specs=[pl.BlockSpec((BLK_, D_), lambda i: (i, 0)),
                  pl.BlockSpec((D_, D_), lambda i: (0, 0)),
                  pl.BlockSpec((1, D_), lambda i: (0, 0)),
                  pl.BlockSpec((1, D_), lambda i: (0, 0))],
        **_HEL_OUT,
    )(x, W, al, ar)


def _combine(acc_ref, s_ref):
    a = acc_ref[0] + acc_ref[1]
    s = s_ref[...] + 1e-9
    return jnp.maximum(a / s, 0.0)


def _tc_mid_body(acc_ref, s_ref, w_ref, al_ref, ar_ref,
                 h_ref, el_ref, er_ref):
    xr = _combine(acc_ref, s_ref)
    h = _dot(xr, w_ref[...])
    h_ref[...] = h
    _proj(h, al_ref, ar_ref, el_ref, er_ref)


def _tc_mid(acc, s_col, W, al, ar):
    return pl.pallas_call(
        _tc_mid_body,
        grid=(N_ // BLK_,),
        in_specs=[pl.BlockSpec((NC_, BLK_, D_), lambda i: (0, i, 0)),
                  pl.BlockSpec((BLK_, 1), lambda i: (i, 0)),
                  pl.BlockSpec((D_, D_), lambda i: (0, 0)),
                  pl.BlockSpec((1, D_), lambda i: (0, 0)),
                  pl.BlockSpec((1, D_), lambda i: (0, 0))],
        **_HEL_OUT,
    )(acc, s_col, W, al, ar)


def _tc_tail_body(acc_ref, s_ref, out_ref):
    out_ref[...] = _combine(acc_ref, s_ref)


def _tc_tail(acc, s_col):
    return pl.pallas_call(
        _tc_tail_body,
        grid=(N_ // BLK_,),
        in_specs=[pl.BlockSpec((NC_, BLK_, D_), lambda i: (0, i, 0)),
                  pl.BlockSpec((BLK_, 1), lambda i: (i, 0))],
        out_specs=pl.BlockSpec((BLK_, D_), lambda i: (i, 0)),
        out_shape=jax.ShapeDtypeStruct((N_, D_), jnp.float32),
    )(acc, s_col)


def _s_col(s2):
    return (s2[0] + s2[1]).reshape(SR_ * D_)[:N_].reshape(N_, 1)


def _sc_edge(h, el, er, src, dst):
    out_type = [jax.ShapeDtypeStruct((NC_, N_, D_), jnp.float32),
                jax.ShapeDtypeStruct((NC_, SR_, D_), jnp.float32)]
    scratch = [
        pltpu.VMEM((2, CHA_), jnp.int32),
        pltpu.VMEM((2, CHB_), jnp.int32),
        pltpu.VMEM((2, CHA_), jnp.int32),
        pltpu.VMEM((2, CHB_), jnp.int32),
        pltpu.VMEM((SR_,), jnp.int32),
        pltpu.VMEM((N_,), jnp.float32),
        pltpu.VMEM((N_,), jnp.float32),
        pltpu.VMEM((SR_, D_), jnp.float32),
        pltpu.VMEM((CHA_, D_), jnp.float32),
        pltpu.VMEM((CHB_, D_), jnp.float32),
        pltpu.VMEM_SHARED((N_, D_), jnp.float32),
        pltpu.VMEM_SHARED((SR_, D_), jnp.float32),
        pltpu.SemaphoreType.DMA,
        pltpu.SemaphoreType.DMA,
        pltpu.SemaphoreType.DMA,
        pltpu.SemaphoreType.DMA,
        pltpu.SemaphoreType.DMA,
        pltpu.SemaphoreType.DMA,
    ]
    mesh = plsc.VectorSubcoreMesh(core_axis_name="c", subcore_axis_name="s")
    cp = pltpu.CompilerParams()
    if "needs_layout_passes" in pltpu.CompilerParams.__dataclass_fields__:
        cp = dataclasses.replace(cp, needs_layout_passes=False)

    @functools.partial(pl.kernel, out_type=out_type, mesh=mesh,
                       scratch_types=scratch, compiler_params=cp)
    def k(h_hbm, el_hbm, er_hbm, src_hbm, dst_hbm, acc_hbm, s2_hbm,
          sidxA, sidxB, didxA, didxB, ident, el_v, er_v, s_loc,
          rows_a, rows_b, out_sh, s2_sh,
          gsemA, gsemB, ssemA, ssemB, isem0, isem1):
        cid = lax.axis_index("c")
        sid = lax.axis_index("s")
        wid = sid * NC_ + cid
        ebase = wid * EPT_

        pltpu.sync_copy(el_hbm, el_v)
        pltpu.sync_copy(er_hbm, er_v)

        zv = jnp.zeros((16,), jnp.float32)

        @pl.loop(0, SR_)
        def _(r):
            for c in range(D_ // 16):
                s_loc[r, pl.ds(c * 16, 16)] = zv

        @pl.loop(0, CHA_)
        def _(r):
            for c in range(D_ // 16):
                rows_a[r, pl.ds(c * 16, 16)] = zv

        @pl.loop(0, CHB_)
        def _(r):
            for c in range(D_ // 16):
                rows_b[r, pl.ds(c * 16, 16)] = zv

        @pl.loop(0, SR_ // 16)
        def _(g):
            ident[pl.ds(g * 16, 16)] = g * 16 + lax.iota(jnp.int32, 16)

        rbase = sid * RPT_
        for z in range(RPT_ // CHA_):
            pltpu.sync_copy(rows_a, out_sh.at[pl.ds(rbase + z * CHA_, CHA_)])

        @pl.when(sid == 0)
        def _():
            pltpu.sync_copy(rows_a.at[pl.ds(0, REM_)],
                            out_sh.at[pl.ds(NS_ * RPT_, REM_)])
            pltpu.sync_copy(s_loc, s2_sh)

        plsc.subcore_barrier()

        pltpu.sync_copy(src_hbm.at[pl.ds(ebase, CHA_)], sidxA.at[0])
        pltpu.sync_copy(src_hbm.at[pl.ds(ebase + CHA_, CHB_)], sidxB.at[0])
        pltpu.sync_copy(dst_hbm.at[pl.ds(ebase, CHA_)], didxA.at[0])
        pltpu.sync_copy(dst_hbm.at[pl.ds(ebase + CHA_, CHB_)], didxB.at[0])

        def compute_w(b):
            ws = []
            for q in range(CH_ // 16):
                if q < CHA_ // 16:
                    s16 = sidxA[b, pl.ds(q * 16, 16)]
                    d16 = didxA[b, pl.ds(q * 16, 16)]
                else:
                    s16 = sidxB[b, pl.ds((q - CHA_ // 16) * 16, 16)]
                    d16 = didxB[b, pl.ds((q - CHA_ // 16) * 16, 16)]
                z = (plsc.load_gather(el_v, [s16])
                     + plsc.load_gather(er_v, [d16]))
                e = jnp.where(z > 0, z, 0.2 * z)
                w = jnp.exp(e)
                ws.append(w)
                plsc.addupdate_scatter(
                    s_loc, [lax.shift_right_logical(d16, 7),
                            jnp.bitwise_and(d16, 127)], w)
            return tuple(ws)

        def scale(buf, wlist):
            for g, wv in enumerate(wlist):
                for j0 in range(16):
                    wb = jnp.full((16,), wv[j0], jnp.float32)
                    r = g * 16 + j0
                    for c in range(D_ // 16):
                        buf[r, pl.ds(c * 16, 16)] = (
                            buf[r, pl.ds(c * 16, 16)] * wb)

        def chunk(i, b, nb, wait_prev, last, ws):
            if wait_prev:
                pltpu.make_async_copy(
                    rows_a, out_sh.at[didxA.at[nb]], ssemA).wait()
                pltpu.make_async_copy(
                    rows_b, out_sh.at[didxB.at[nb]], ssemB).wait()
            ga = pltpu.async_copy(h_hbm.at[sidxA.at[b]], rows_a, gsemA)
            gb = pltpu.async_copy(h_hbm.at[sidxB.at[b]], rows_b, gsemB)
            if not last:
                off = ebase + (i + 1) * CH_
                ic0 = pltpu.async_copy(src_hbm.at[pl.ds(off, CHA_)],
                                       sidxA.at[nb], isem0)
                ic1 = pltpu.async_copy(src_hbm.at[pl.ds(off + CHA_, CHB_)],
                                       sidxB.at[nb], isem0)
                ic2 = pltpu.async_copy(dst_hbm.at[pl.ds(off, CHA_)],
                                       didxA.at[nb], isem1)
                ic3 = pltpu.async_copy(dst_hbm.at[pl.ds(off + CHA_, CHB_)],
                                       didxB.at[nb], isem1)
            ga.wait()
            scale(rows_a, ws[:CHA_ // 16])
            sca = pltpu.async_copy(rows_a, out_sh.at[didxA.at[b]], ssemA,
                                   add=True)
            gb.wait()
            scale(rows_b, ws[CHA_ // 16:])
            scb = pltpu.async_copy(rows_b, out_sh.at[didxB.at[b]], ssemB,
                                   add=True)
            if last:
                sca.wait()
                scb.wait()
                return None
            ic0.wait()
            ic1.wait()
            ic2.wait()
            ic3.wait()
            return compute_w(nb)

        wsc = compute_w(0)
        wsc = chunk(0, 0, 1, False, False, wsc)

        @pl.loop(1, NCHUNK_ - 3, step=2, init_carry=wsc)
        def wsc(i, carry):
            carry = chunk(i, 1, 0, True, False, carry)
            carry = chunk(i + 1, 0, 1, True, False, carry)
            return carry

        wsc = chunk(NCHUNK_ - 2, 1, 0, True, False, wsc)
        chunk(NCHUNK_ - 1, 0, 1, True, True, wsc)

        pltpu.sync_copy(s_loc, s2_sh.at[ident], add=True)

        plsc.subcore_barrier()
        for z in range(RPT_ // ZR_):
            r0 = rbase + z * ZR_
            pltpu.sync_copy(out_sh.at[pl.ds(r0, ZR_)],
                            acc_hbm.at[cid].at[pl.ds(r0, ZR_)])

        @pl.when(sid == 0)
        def _():
            r0 = NS_ * RPT_
            pltpu.sync_copy(out_sh.at[pl.ds(r0, REM_)],
                            acc_hbm.at[cid].at[pl.ds(r0, REM_)])
            pltpu.sync_copy(s2_sh, s2_hbm.at[cid])

    return k(h, el, er, src, dst)


@jax.jit
def kernel(x, edge_index, W1, al1, ar1, W2, al2, ar2, W3, al3, ar3):
    src = edge_index[0]
    dst = edge_index[1]

    h, el, er = _tc_head(x, W1, al1.reshape(1, D_), ar1.reshape(1, D_))
    acc, s2 = _sc_edge(h, el.reshape(N_), er.reshape(N_), src, dst)
    h, el, er = _tc_mid(acc, _s_col(s2), W2, al2.reshape(1, D_),
                        ar2.reshape(1, D_))
    acc, s2 = _sc_edge(h, el.reshape(N_), er.reshape(N_), src, dst)
    h, el, er = _tc_mid(acc, _s_col(s2), W3, al3.reshape(1, D_),
                        ar3.reshape(1, D_))
    acc, s2 = _sc_edge(h, el.reshape(N_), er.reshape(N_), src, dst)
    return _tc_tail(acc, _s_col(s2))


# speedup vs baseline: 1.0225x; 1.0225x over previous
import dataclasses
import functools

import jax
import jax.numpy as jnp
from jax import lax
from jax.experimental import pallas as pl
from jax.experimental.pallas import tpu as pltpu
from jax.experimental.pallas import tpu_sc as plsc

N_ = 10000
D_ = 128
E_ = 320000
NC_ = 2
NS_ = 16
NW_ = NC_ * NS_
EPT_ = E_ // NW_
CH_ = 80
CHA_ = 48
CHB_ = 32
NCHUNK_ = EPT_ // CH_
RPT_ = 624
REM_ = N_ - NS_ * RPT_
ZR_ = 208
SR_ = 80
BLK_ = 2000


def _dot(a, b):
    return lax.dot_general(a, b, (((1,), (0,)), ((), ())),
                           precision=lax.Precision.HIGHEST)


def _proj(h, al_ref, ar_ref, el_ref, er_ref):
    el_ref[...] = jnp.sum(h * al_ref[...], axis=1, keepdims=True)
    er_ref[...] = jnp.sum(h * ar_ref[...], axis=1, keepdims=True)


def _tc_head_body(x_ref, w_ref, al_ref, ar_ref, h_ref, el_ref, er_ref):
    h = _dot(x_ref[...], w_ref[...])
    h_ref[...] = h
    _proj(h, al_ref, ar_ref, el_ref, er_ref)


_HEL_OUT = dict(
    out_specs=[pl.BlockSpec((BLK_, D_), lambda i: (i, 0)),
               pl.BlockSpec((BLK_, 1), lambda i: (i, 0)),
               pl.BlockSpec((BLK_, 1), lambda i: (i, 0))],
    out_shape=[jax.ShapeDtypeStruct((N_, D_), jnp.float32),
               jax.ShapeDtypeStruct((N_, 1), jnp.float32),
               jax.ShapeDtypeStruct((N_, 1), jnp.float32)],
)


def _tc_head(x, W, al, ar):
    return pl.pallas_call(
        _tc_head_body,
        grid=(N_ // BLK_,),
        in_specs=[pl.BlockSpec((BLK_, D_), lambda i: (i, 0)),
                  pl.BlockSpec((D_, D_), lambda i: (0, 0)),
                  pl.BlockSpec((1, D_), lambda i: (0, 0)),
                  pl.BlockSpec((1, D_), lambda i: (0, 0))],
        **_HEL_OUT,
    )(x, W, al, ar)


def _combine(acc_ref, s_ref):
    a = acc_ref[0] + acc_ref[1]
    s = s_ref[...] + 1e-9
    return jnp.maximum(a / s, 0.0)


def _tc_mid_body(acc_ref, s_ref, w_ref, al_ref, ar_ref,
                 h_ref, el_ref, er_ref):
    xr = _combine(acc_ref, s_ref)
    h = _dot(xr, w_ref[...])
    h_ref[...] = h
    _proj(h, al_ref, ar_ref, el_ref, er_ref)


def _tc_mid(acc, s_col, W, al, ar):
    return pl.pallas_call(
        _tc_mid_body,
        grid=(N_ // BLK_,),
        in_specs=[pl.BlockSpec((NC_, BLK_, D_), lambda i: (0, i, 0)),
                  pl.BlockSpec((BLK_, 1), lambda i: (i, 0)),
                  pl.BlockSpec((D_, D_), lambda i: (0, 0)),
                  pl.BlockSpec((1, D_), lambda i: (0, 0)),
                  pl.BlockSpec((1, D_), lambda i: (0, 0))],
        **_HEL_OUT,
    )(acc, s_col, W, al, ar)


def _tc_tail_body(acc_ref, s_ref, out_ref):
    out_ref[...] = _combine(acc_ref, s_ref)


def _tc_tail(acc, s_col):
    return pl.pallas_call(
        _tc_tail_body,
        grid=(N_ // BLK_,),
        in_specs=[pl.BlockSpec((NC_, BLK_, D_), lambda i: (0, i, 0)),
                  pl.BlockSpec((BLK_, 1), lambda i: (i, 0))],
        out_specs=pl.BlockSpec((BLK_, D_), lambda i: (i, 0)),
        out_shape=jax.ShapeDtypeStruct((N_, D_), jnp.float32),
    )(acc, s_col)


def _s_col(s2):
    return (s2[0] + s2[1]).reshape(SR_ * D_)[:N_].reshape(N_, 1)


def _sc_edge(h, el, er, src, dst):
    out_type = [jax.ShapeDtypeStruct((NC_, N_, D_), jnp.float32),
                jax.ShapeDtypeStruct((NC_, SR_, D_), jnp.float32)]
    scratch = [
        pltpu.VMEM((2, CHA_), jnp.int32),
        pltpu.VMEM((2, CHB_), jnp.int32),
        pltpu.VMEM((2, CHA_), jnp.int32),
        pltpu.VMEM((2, CHB_), jnp.int32),
        pltpu.VMEM((SR_,), jnp.int32),
        pltpu.VMEM((N_,), jnp.float32),
        pltpu.VMEM((N_,), jnp.float32),
        pltpu.VMEM((SR_, D_), jnp.float32),
        pltpu.VMEM((CHA_, D_), jnp.float32),
        pltpu.VMEM((CHB_, D_), jnp.float32),
        pltpu.VMEM_SHARED((N_, D_), jnp.float32),
        pltpu.VMEM_SHARED((SR_, D_), jnp.float32),
        pltpu.SemaphoreType.DMA,
        pltpu.SemaphoreType.DMA,
        pltpu.SemaphoreType.DMA,
        pltpu.SemaphoreType.DMA,
        pltpu.SemaphoreType.DMA,
        pltpu.SemaphoreType.DMA,
    ]
    mesh = plsc.VectorSubcoreMesh(core_axis_name="c", subcore_axis_name="s")
    cp = pltpu.CompilerParams()
    if "needs_layout_passes" in pltpu.CompilerParams.__dataclass_fields__:
        cp = dataclasses.replace(cp, needs_layout_passes=False)

    @functools.partial(pl.kernel, out_type=out_type, mesh=mesh,
                       scratch_types=scratch, compiler_params=cp)
    def k(h_hbm, el_hbm, er_hbm, src_hbm, dst_hbm, acc_hbm, s2_hbm,
          sidxA, sidxB, didxA, didxB, ident, el_v, er_v, s_loc,
          rows_a, rows_b, out_sh, s2_sh,
          gsemA, gsemB, ssemA, ssemB, isem0, isem1):
        cid = lax.axis_index("c")
        sid = lax.axis_index("s")
        wid = sid * NC_ + cid
        ebase = wid * EPT_

        pltpu.sync_copy(el_hbm, el_v)
        pltpu.sync_copy(er_hbm, er_v)

        zv = jnp.zeros((16,), jnp.float32)

        @pl.loop(0, SR_)
        def _(r):
            for c in range(D_ // 16):
                s_loc[r, pl.ds(c * 16, 16)] = zv

        @pl.loop(0, CHA_)
        def _(r):
            for c in range(D_ // 16):
                rows_a[r, pl.ds(c * 16, 16)] = zv

        @pl.loop(0, CHB_)
        def _(r):
            for c in range(D_ // 16):
                rows_b[r, pl.ds(c * 16, 16)] = zv

        @pl.loop(0, SR_ // 16)
        def _(g):
            ident[pl.ds(g * 16, 16)] = g * 16 + lax.iota(jnp.int32, 16)

        rbase = sid * RPT_
        for z in range(RPT_ // CHA_):
            pltpu.sync_copy(rows_a, out_sh.at[pl.ds(rbase + z * CHA_, CHA_)])

        @pl.when(sid == 0)
        def _():
            pltpu.sync_copy(rows_a.at[pl.ds(0, REM_)],
                            out_sh.at[pl.ds(NS_ * RPT_, REM_)])
            pltpu.sync_copy(s_loc, s2_sh)

        plsc.subcore_barrier()

        pltpu.sync_copy(src_hbm.at[pl.ds(ebase, CHA_)], sidxA.at[0])
        pltpu.sync_copy(src_hbm.at[pl.ds(ebase + CHA_, CHB_)], sidxB.at[0])
        pltpu.sync_copy(dst_hbm.at[pl.ds(ebase, CHA_)], didxA.at[0])
        pltpu.sync_copy(dst_hbm.at[pl.ds(ebase + CHA_, CHB_)], didxB.at[0])

        def chunk(i, b, nb, wait_prev, last):
            if wait_prev:
                pltpu.make_async_copy(
                    rows_a, out_sh.at[didxA.at[nb]], ssemA).wait()
                pltpu.make_async_copy(
                    rows_b, out_sh.at[didxB.at[nb]], ssemB).wait()
            ga = pltpu.async_copy(h_hbm.at[sidxA.at[b]], rows_a, gsemA)
            gb = pltpu.async_copy(h_hbm.at[sidxB.at[b]], rows_b, gsemB)
            if not last:
                off = ebase + (i + 1) * CH_
                ic0 = pltpu.async_copy(src_hbm.at[pl.ds(off, CHA_)],
                                       sidxA.at[nb], isem0)
                ic1 = pltpu.async_copy(src_hbm.at[pl.ds(off + CHA_, CHB_)],
                                       sidxB.at[nb], isem0)
                ic2 = pltpu.async_copy(dst_hbm.at[pl.ds(off, CHA_)],
                                       didxA.at[nb], isem1)
                ic3 = pltpu.async_copy(dst_hbm.at[pl.ds(off + CHA_, CHB_)],
                                       didxB.at[nb], isem1)
            ws = []
            for q in range(CH_ // 16):
                if q < CHA_ // 16:
                    s16 = sidxA[b, pl.ds(q * 16, 16)]
                    d16 = didxA[b, pl.ds(q * 16, 16)]
                else:
                    s16 = sidxB[b, pl.ds((q - CHA_ // 16) * 16, 16)]
                    d16 = didxB[b, pl.ds((q - CHA_ // 16) * 16, 16)]
                z = (plsc.load_gather(el_v, [s16])
                     + plsc.load_gather(er_v, [d16]))
                e = jnp.where(z > 0, z, 0.2 * z)
                w = jnp.exp(e)
                ws.append(w)
                plsc.addupdate_scatter(
                    s_loc, [lax.shift_right_logical(d16, 7),
                            jnp.bitwise_and(d16, 127)], w)

            def scale(buf, wlist, base):
                for g, wv in enumerate(wlist):
                    for j0 in range(16):
                        wb = jnp.full((16,), wv[j0], jnp.float32)
                        r = g * 16 + j0
                        for c in range(D_ // 16):
                            buf[r, pl.ds(c * 16, 16)] = (
                                buf[r, pl.ds(c * 16, 16)] * wb)

            ga.wait()
            scale(rows_a, ws[:CHA_ // 16], 0)
            sca = pltpu.async_copy(rows_a, out_sh.at[didxA.at[b]], ssemA,
                                   add=True)
            gb.wait()
            scale(rows_b, ws[CHA_ // 16:], CHA_)
            scb = pltpu.async_copy(rows_b, out_sh.at[didxB.at[b]], ssemB,
                                   add=True)
            if last:
                sca.wait()
                scb.wait()
            if not last:
                ic0.wait()
                ic1.wait()
                ic2.wait()
                ic3.wait()

        chunk(0, 0, 1, False, False)

        @pl.loop(1, NCHUNK_ - 3, step=2)
        def _(i):
            chunk(i, 1, 0, True, False)
            chunk(i + 1, 0, 1, True, False)

        chunk(NCHUNK_ - 2, 1, 0, True, False)
        chunk(NCHUNK_ - 1, 0, 1, True, True)

        pltpu.sync_copy(s_loc, s2_sh.at[ident], add=True)

        plsc.subcore_barrier()
        for z in range(RPT_ // ZR_):
            r0 = rbase + z * ZR_
            pltpu.sync_copy(out_sh.at[pl.ds(r0, ZR_)],
                            acc_hbm.at[cid].at[pl.ds(r0, ZR_)])

        @pl.when(sid == 0)
        def _():
            r0 = NS_ * RPT_
            pltpu.sync_copy(out_sh.at[pl.ds(r0, REM_)],
                            acc_hbm.at[cid].at[pl.ds(r0, REM_)])
            pltpu.sync_copy(s2_sh, s2_hbm.at[cid])

    return k(h, el, er, src, dst)


@jax.jit
def kernel(x, edge_index, W1, al1, ar1, W2, al2, ar2, W3, al3, ar3):
    src = edge_index[0]
    dst = edge_index[1]

    h, el, er = _tc_head(x, W1, al1.reshape(1, D_), ar1.reshape(1, D_))
    acc, s2 = _sc_edge(h, el.reshape(N_), er.reshape(N_), src, dst)
    h, el, er = _tc_mid(acc, _s_col(s2), W2, al2.reshape(1, D_),
                        ar2.reshape(1, D_))
    acc, s2 = _sc_edge(h, el.reshape(N_), er.reshape(N_), src, dst)
    h, el, er = _tc_mid(acc, _s_col(s2), W3, al3.reshape(1, D_),
                        ar3.reshape(1, D_))
    acc, s2 = _sc_edge(h, el.reshape(N_), er.reshape(N_), src, dst)
    return _tc_tail(acc, _s_col(s2))
